# baseline (speedup 1.0000x reference)
import numpy as np
import jax
import jax.numpy as jnp
from jax.experimental import pallas as pl

_NBATCH = 2; _NCOIL = 8; _KLEN = 32768
_IM = (256, 256); _GRID = (512, 512)
_J = (6, 6); _L = (1024, 1024)
_NSHIFT = (128.0, 128.0)


def kernel(x, smap, om, scaling_coef, table0, table1):
    xc = x[:, :, 0] + 1j * x[:, :, 1]
    sc = smap[:, :, 0] + 1j * smap[:, :, 1]
    xs = xc * sc
    xs = xs * scaling_coef[None, None].astype(xs.dtype)
    xg = jnp.pad(xs, ((0, 0), (0, 0), (0, _GRID[0] - _IM[0]), (0, _GRID[1] - _IM[1])))
    kg = jnp.fft.fftn(xg, axes=(-2, -1))
    kgf = kg.reshape(_NBATCH, _NCOIL, _GRID[0] * _GRID[1])
    gsz = jnp.array(_GRID, dtype=om.dtype)
    tm = om * (gsz[None, :, None] / (2.0 * np.pi))
    base = jnp.floor(tm - jnp.array(_J, dtype=om.dtype)[None, :, None] / 2.0)
    J0, J1 = _J
    L0, L1 = _L
    c0 = L0 * J0 // 2
    c1 = L1 * J1 // 2
    out = jnp.zeros((_NBATCH, _NCOIL, _KLEN), dtype=kg.dtype)
    for j0 in range(J0):
        gi0 = base[:, 0] + (j0 + 1)
        d0 = jnp.around((tm[:, 0] - gi0) * L0).astype(jnp.int32) + c0
        coef0 = jnp.take(table0, d0)
        i0 = jnp.mod(gi0, _GRID[0]).astype(jnp.int32)
        for j1 in range(J1):
            gi1 = base[:, 1] + (j1 + 1)
            d1 = jnp.around((tm[:, 1] - gi1) * L1).astype(jnp.int32) + c1
            coef = coef0 * jnp.take(table1, d1)
            flat = i0 * _GRID[1] + jnp.mod(gi1, _GRID[1]).astype(jnp.int32)
            vals = jnp.take_along_axis(
                kgf, jnp.broadcast_to(flat[:, None, :], (_NBATCH, _NCOIL, _KLEN)), axis=2)
            out = out + coef[:, None, :].astype(kg.dtype) * vals
    phase = jnp.exp(1j * jnp.einsum('bdk,d->bk', om, jnp.array(_NSHIFT, dtype=om.dtype)))
    out = out * phase[:, None, :].astype(out.dtype)
    return jnp.stack((jnp.real(out), jnp.imag(out)), axis=2).astype(jnp.float32)


# speedup vs baseline: 10.2291x; 10.2291x over previous
import dataclasses
import functools
import numpy as np
import jax
import jax.numpy as jnp
from jax import lax
from jax.experimental import pallas as pl
from jax.experimental.pallas import tpu as pltpu
from jax.experimental.pallas import tpu_sc as plsc

_B = 2
_C = 8
_K = 32768
_BK = _B * _K
_IM = 256
_GR = 512
_J = 6
_L = 1024
_CTR = _L * _J // 2
_NW = 32
_PTS = _BK // _NW
_GSZ = 16
_NG = _PTS // _GSZ
_NTAP = _J * _J
_TBL_PAD = 6152
_NSHIFT = (128.0, 128.0)

_m = np.arange(_GR)[:, None]
_n = np.arange(_IM)[None, :]
_ang = -2.0 * np.pi * (_m * _n) / float(_GR)
_FR = np.cos(_ang).astype(np.float32)
_FI = np.sin(_ang).astype(np.float32)


def _grid_body(x_ref, s_ref, ap_ref, fr_ref, fi_ref, o_ref):
    xr = x_ref[0, 0]
    xi = x_ref[0, 1]
    sr = s_ref[0, 0, 0]
    si = s_ref[0, 0, 1]
    ap = ap_ref[...]
    Xr = (xr * sr - xi * si) * ap
    Xi = (xr * si + xi * sr) * ap
    fr = fr_ref[...]
    fi = fi_ref[...]
    dot = functools.partial(
        lax.dot_general,
        dimension_numbers=(((1,), (0,)), ((), ())),
        preferred_element_type=jnp.float32,
        precision=lax.Precision.HIGHEST,
    )
    dott = functools.partial(
        lax.dot_general,
        dimension_numbers=(((1,), (1,)), ((), ())),
        preferred_element_type=jnp.float32,
        precision=lax.Precision.HIGHEST,
    )
    Ar = dot(fr, Xr) - dot(fi, Xi)
    Ai = dot(fr, Xi) + dot(fi, Xr)
    o_ref[0, 0, 0] = dott(Ar, fr) - dott(Ai, fi)
    o_ref[0, 0, 1] = dott(Ar, fi) + dott(Ai, fr)


def _build_grid(x, smap, scaling_coef, frv, fiv):
    return pl.pallas_call(
        _grid_body,
        grid=(_B, _C),
        in_specs=[
            pl.BlockSpec((1, 2, _IM, _IM), lambda b, c: (b, 0, 0, 0)),
            pl.BlockSpec((1, 1, 2, _IM, _IM), lambda b, c: (b, c, 0, 0, 0)),
            pl.BlockSpec((_IM, _IM), lambda b, c: (0, 0)),
            pl.BlockSpec((_GR, _IM), lambda b, c: (0, 0)),
            pl.BlockSpec((_GR, _IM), lambda b, c: (0, 0)),
        ],
        out_specs=pl.BlockSpec((1, 1, 2, _GR, _GR), lambda b, c: (b, c, 0, 0, 0)),
        out_shape=jax.ShapeDtypeStruct((_B, _C, 2, _GR, _GR), jnp.float32),
    )(x, smap, scaling_coef, frv, fiv)


_WB = 2048
_SCALE = float(np.float32(_GR) / np.float32(2.0 * np.pi))


def _prep_body(om_ref, i0_ref, i1_ref, d0_ref, d1_ref):
    b = pl.program_id(0) // (_K // _WB)
    om = om_ref[0]
    tm = om * _SCALE
    base = jnp.floor(tm - _J / 2.0)
    for j in range(_J):
        gi = base + (j + 1.0)
        d = jnp.around((tm - gi) * float(_L)).astype(jnp.int32) + _CTR
        ii = jnp.mod(gi, float(_GR)).astype(jnp.int32)
        i0_ref[j, :] = b * (_GR * _GR) + ii[0] * _GR
        i1_ref[j, :] = ii[1]
        d0_ref[j, :] = d[0]
        d1_ref[j, :] = d[1]


def _build_prep(om):
    nsteps = _BK // _WB
    out_shapes = [
        jax.ShapeDtypeStruct((_J, _BK), jnp.int32),
        jax.ShapeDtypeStruct((_J, _BK), jnp.int32),
        jax.ShapeDtypeStruct((_J, _BK), jnp.int32),
        jax.ShapeDtypeStruct((_J, _BK), jnp.int32),
    ]
    col_spec6 = pl.BlockSpec((_J, _WB), lambda i: (0, i))
    return pl.pallas_call(
        _prep_body,
        grid=(nsteps,),
        in_specs=[pl.BlockSpec((1, 2, _WB), lambda i: (i // (_K // _WB), 0, i % (_K // _WB)))],
        out_specs=[col_spec6, col_spec6, col_spec6, col_spec6],
        out_shape=out_shapes,
    )(om)


def _sc_body(g_hbm, i0_hbm, i1_hbm, d0_hbm, d1_hbm, ph_hbm, t0_hbm, t1_hbm,
             out_hbm,
             t0_v, t1_v, i0_v, i1_v, d0_v, d1_v, ph_v,
             gidx_v, rows_v, cf_v, acc_v, accT_v, sem):
    wid = lax.axis_index("s") * 2 + lax.axis_index("c")
    b = wid // (_NW // _B)
    colbase = wid * _PTS
    kbase = colbase - b * _K

    pltpu.sync_copy(t0_hbm, t0_v)
    pltpu.sync_copy(t1_hbm, t1_v)
    pltpu.sync_copy(i0_hbm.at[:, pl.ds(colbase, _PTS)], i0_v)
    pltpu.sync_copy(i1_hbm.at[:, pl.ds(colbase, _PTS)], i1_v)
    pltpu.sync_copy(d0_hbm.at[:, pl.ds(colbase, _PTS)], d0_v)
    pltpu.sync_copy(d1_hbm.at[:, pl.ds(colbase, _PTS)], d1_v)
    pltpu.sync_copy(ph_hbm.at[:, pl.ds(colbase, _PTS)], ph_v)

    lane = lax.iota(jnp.int32, 16)
    lane36 = lane * _NTAP
    lane_x = lane ^ 1
    sgn = jnp.where((lane & 1) == 1, 1.0, -1.0).astype(jnp.float32)
    zero16 = jnp.zeros((16,), jnp.int32)
    one16 = zero16 + 1

    @pl.loop(0, _NG // 8)
    def _ogroup(og):
        @pl.loop(0, 8)
        def _group(sg):
            g = og * 8 + sg
            p0 = g * _GSZ
            c1s = []
            for j1 in range(_J):
                c1s.append(plsc.load_gather(t1_v, [d1_v[j1, pl.ds(p0, 16)]]))
            for j0 in range(_J):
                a0 = i0_v[j0, pl.ds(p0, 16)]
                c0 = plsc.load_gather(t0_v, [d0_v[j0, pl.ds(p0, 16)]])
                for j1 in range(_J):
                    t = j0 * _J + j1
                    fl = a0 + i1_v[j1, pl.ds(p0, 16)]
                    plsc.store_scatter(gidx_v, [lane36 + t], fl)
                    cf_v[t, :] = c0 * c1s[j1]
            cps = [
                pltpu.async_copy(
                    g_hbm.at[gidx_v.at[pl.ds(s * 96, 96)]],
                    rows_v.at[pl.ds(s * 96, 96)], sem)
                for s in range(6)
            ]
            for cp in cps:
                cp.wait()

            @pl.loop(0, _GSZ)
            def _point(p):
                fullp = jnp.full((16,), p, jnp.int32)
                p36 = p * _NTAP

                def tap(t, acc):
                    cf = plsc.load_gather(cf_v, [jnp.full((16,), t, jnp.int32), fullp])
                    rw = plsc.load_gather(rows_v, [jnp.full((16,), p36 + t, jnp.int32), lane])
                    return acc + cf * rw

                acc = lax.fori_loop(0, _NTAP, tap, jnp.zeros((16,), jnp.float32))
                plsc.store_scatter(acc_v, [fullp, lane], acc)
                sw = plsc.load_gather(acc_v, [fullp, lane_x])
                phcol = jnp.full((16,), p0 + p, jnp.int32)
                pr = plsc.load_gather(ph_v, [zero16, phcol])
                pi = plsc.load_gather(ph_v, [one16, phcol])
                res = acc * pr + sw * pi * sgn
                plsc.store_scatter(accT_v, [lane, jnp.full((16,), sg * _GSZ, jnp.int32) + fullp], res)

        pltpu.sync_copy(accT_v, out_hbm.at[b, :, pl.ds(kbase + og * 128, 128)])


def _build_sc(g, i0, i1, d0, d1, ph, t0, t1):
    mesh = plsc.VectorSubcoreMesh(core_axis_name="c", subcore_axis_name="s")
    cp = pltpu.CompilerParams()
    if "needs_layout_passes" in pltpu.CompilerParams.__dataclass_fields__:
        cp = dataclasses.replace(cp, needs_layout_passes=False)
    if "use_tc_tiling_on_sc" in pltpu.CompilerParams.__dataclass_fields__:
        cp = dataclasses.replace(cp, use_tc_tiling_on_sc=False)
    kfn = pl.kernel(
        _sc_body,
        out_type=jax.ShapeDtypeStruct((_B, 16, _K), jnp.float32),
        mesh=mesh,
        compiler_params=cp,
        scratch_types=[
            pltpu.VMEM((_TBL_PAD,), jnp.float32),
            pltpu.VMEM((_TBL_PAD,), jnp.float32),
            pltpu.VMEM((_J, _PTS), jnp.int32),
            pltpu.VMEM((_J, _PTS), jnp.int32),
            pltpu.VMEM((_J, _PTS), jnp.int32),
            pltpu.VMEM((_J, _PTS), jnp.int32),
            pltpu.VMEM((2, _PTS), jnp.float32),
            pltpu.VMEM((_NTAP * _GSZ,), jnp.int32),
            pltpu.VMEM((_NTAP * _GSZ, 16), jnp.float32),
            pltpu.VMEM((_NTAP, 16), jnp.float32),
            pltpu.VMEM((16, 16), jnp.float32),
            pltpu.VMEM((16, 128), jnp.float32),
            pltpu.SemaphoreType.DMA,
        ],
    )
    return kfn(g, i0, i1, d0, d1, ph, t0, t1)


def kernel(x, smap, om, scaling_coef, table0, table1):
    xs = x.reshape(_B, 2, _IM, _IM)
    frv = jnp.asarray(_FR)
    fiv = jnp.asarray(_FI)
    gp = _build_grid(xs, smap, scaling_coef, frv, fiv)
    g = jnp.transpose(gp, (0, 3, 4, 1, 2)).reshape(_B * _GR * _GR, 16)
    i0, i1, d0, d1 = _build_prep(om)
    th = jnp.einsum('bdk,d->bk', om, jnp.array(_NSHIFT, dtype=om.dtype))
    ph = jnp.stack([jnp.cos(th), jnp.sin(th)]).reshape(2, _BK)
    t0p = jnp.concatenate([table0, jnp.zeros((_TBL_PAD - table0.shape[0],), jnp.float32)])
    t1p = jnp.concatenate([table1, jnp.zeros((_TBL_PAD - table1.shape[0],), jnp.float32)])
    out = _build_sc(g, i0, i1, d0, d1, ph, t0p, t1p)
    return out.reshape(_B, _C, 2, _K)
